# baseline (speedup 1.0000x reference)
import functools

import jax
import jax.numpy as jnp
import numpy as np
from jax import lax
from jax.experimental import pallas as pl
from jax.experimental.pallas import tpu as pltpu

_B, _D, _L, _Q, _E, _NB, _NI = 32, 16, 512, 8, 128, 30, 100000
_BN_EPS = 1e-3
_LO, _HI = 0.001, 1.0
_BN_INV = np.float32(1.0 / np.sqrt(1.0 + _BN_EPS))
_IDF_ROWS = (_NI + 127) // 128


def _drmm_kernel(q_sm, emb_ref, idf_ref, doc_ref, wg_ref, gin_ref, bin_ref,
                 w0_ref, b0_ref, g0_ref, be0_ref,
                 w1_ref, b1_ref, g1_ref, be1_ref,
                 w2_ref, b2_ref, g2_ref, be2_ref,
                 w3_ref, b3_ref, g3_ref, be3_ref,
                 out_ref,
                 doc_sm, qv_s, idfm_s, tile_s, x_s, sem):
    b = pl.program_id(0)

    cp = pltpu.make_async_copy(doc_ref.at[0], doc_sm, sem)
    cp.start()

    lane = lax.broadcasted_iota(jnp.int32, (1, 128), 1)
    for qi in range(_Q):
        qidx = q_sm[b * _Q + qi]
        qv_s[pl.ds(qi, 1), :] = emb_ref[qidx]
        row = idf_ref[qidx >> 7]
        idfm_s[pl.ds(qi, 1), :] = row * (lane == (qidx & 127)).astype(jnp.float32)

    qv = qv_s[...]
    qn = jnp.sqrt(jnp.sum(qv * qv, axis=1, keepdims=True))
    q_idf = jnp.sum(idfm_s[...], axis=1, keepdims=True)

    z = wg_ref[...] * q_idf
    z = z - jnp.max(z, axis=0, keepdims=True)
    ez = jnp.exp(z)
    gate = ez / jnp.sum(ez, axis=0, keepdims=True)

    cp.wait()

    ones_row = jnp.ones((1, 128), jnp.float32)

    def per_passage(dd, carry):
        doff = dd * _L

        for u in range(_L):
            tile_s[u] = emb_ref[doc_sm[0, doff + u]]

        tile = tile_s[...].reshape(_L, _E)
        dn_dims = (((1,), (1,)), ((), ()))
        dots = lax.dot_general(qv, tile, dn_dims,
                               preferred_element_type=jnp.float32)
        n2 = lax.dot_general(ones_row, tile * tile, dn_dims,
                             precision=lax.Precision.HIGHEST,
                             preferred_element_type=jnp.float32)
        denom = qn * jnp.sqrt(n2) + 1e-8
        sims = dots / denom

        bf = jnp.floor((sims - _LO) / (_HI - _LO) * _NB)
        bf = jnp.clip(bf, 0.0, float(_NB - 1))

        for nb in range(_NB):
            cnt = jnp.sum((bf == float(nb)).astype(jnp.float32),
                          axis=1, keepdims=True)
            x_s[pl.ds(dd * _Q, _Q), pl.ds(nb, 1)] = cnt
        return carry

    lax.fori_loop(0, _D, per_passage, 0)

    x = x_s[...]
    x = gin_ref[...] * (x * _BN_INV) + bin_ref[...]
    for w_r, b_r, g_r, be_r in ((w0_ref, b0_ref, g0_ref, be0_ref),
                                (w1_ref, b1_ref, g1_ref, be1_ref),
                                (w2_ref, b2_ref, g2_ref, be2_ref),
                                (w3_ref, b3_ref, g3_ref, be3_ref)):
        y = jnp.dot(x, w_r[...], preferred_element_type=jnp.float32) + b_r[...]
        x = jnp.tanh(g_r[...] * (y * _BN_INV) + be_r[...])

    qout = x
    gate_t = jnp.concatenate([gate] * _D, axis=0)
    prod = qout * gate_t
    si = lax.broadcasted_iota(jnp.int32, (128, _D), 0)
    ci = lax.broadcasted_iota(jnp.int32, (128, _D), 1)
    pm = ((si >> 3) == ci).astype(jnp.float32)
    out = lax.dot_general(prod, pm, (((0,), (0,)), ((), ())),
                          preferred_element_type=jnp.float32)
    out_ref[0] = out


def _whole(shape):
    zeros = (0,) * len(shape)
    return pl.BlockSpec(shape, lambda b, *_: zeros)


@jax.jit
def kernel(doc, query, item_embedding, idf_table, w_gate, g_in, b_in,
           W0, b0, g0, be0, W1, b1, g1, be1, W2, b2, g2, be2, W3, b3, g3, be3):
    emb3 = item_embedding.reshape(_NI, 1, _E)
    idf_flat = jnp.pad(idf_table[:, 0], (0, _IDF_ROWS * 128 - _NI))
    idf3 = idf_flat.reshape(_IDF_ROWS, 1, 128)
    doc3 = doc.reshape(_B, 1, _D * _L)
    qflat = query.reshape(-1)

    row1 = lambda a: a.reshape(1, -1)
    params = (emb3, idf3, doc3, w_gate.reshape(_Q, 1),
              row1(g_in), row1(b_in),
              W0, row1(b0), row1(g0), row1(be0),
              W1, row1(b1), row1(g1), row1(be1),
              W2, row1(b2), row1(g2), row1(be2),
              W3, row1(b3), row1(g3), row1(be3))

    in_specs = [
        _whole((_NI, 1, _E)),
        _whole((_IDF_ROWS, 1, 128)),
        pl.BlockSpec((1, 1, _D * _L), lambda b, *_: (b, 0, 0)),
    ] + [_whole(p.shape) for p in params[3:]]

    grid_spec = pltpu.PrefetchScalarGridSpec(
        num_scalar_prefetch=1,
        grid=(_B,),
        in_specs=in_specs,
        out_specs=pl.BlockSpec((1, 1, _D), lambda b, *_: (b, 0, 0)),
        scratch_shapes=[
            pltpu.SMEM((1, _D * _L), jnp.int32),
            pltpu.VMEM((_Q, _E), jnp.float32),
            pltpu.VMEM((_Q, _E), jnp.float32),
            pltpu.VMEM((_L, 1, _E), jnp.float32),
            pltpu.VMEM((_D * _Q, _NB), jnp.float32),
            pltpu.SemaphoreType.DMA,
        ],
    )

    out = pl.pallas_call(
        _drmm_kernel,
        grid_spec=grid_spec,
        out_shape=jax.ShapeDtypeStruct((_B, 1, _D), jnp.float32),
        compiler_params=pltpu.CompilerParams(
            dimension_semantics=("parallel",),
            vmem_limit_bytes=56 * 1024 * 1024,
        ),
    )(qflat, *params)
    return out.reshape(_B, _D)


# speedup vs baseline: 1.4822x; 1.4822x over previous
import functools

import jax
import jax.numpy as jnp
import numpy as np
from jax import lax
from jax.experimental import pallas as pl
from jax.experimental.pallas import tpu as pltpu

_B, _D, _L, _Q, _E, _NB, _NI = 32, 16, 512, 8, 128, 30, 100000
_BN_EPS = 1e-3
_LO, _HI = 0.001, 1.0
_BN_INV = np.float32(1.0 / np.sqrt(1.0 + _BN_EPS))
_IDF_ROWS = (_NI + 127) // 128


def _drmm_kernel(q_sm, emb_ref, idf_ref, doc_ref, wg_ref, gin_ref, bin_ref,
                 w0_ref, b0_ref, g0_ref, be0_ref,
                 w1_ref, b1_ref, g1_ref, be1_ref,
                 w2_ref, b2_ref, g2_ref, be2_ref,
                 w3_ref, b3_ref, g3_ref, be3_ref,
                 out_ref,
                 doc_sm, qv_s, idfm_s, tile_s, x_s, sem):
    b = pl.program_id(0)

    cp = pltpu.make_async_copy(doc_ref.at[0], doc_sm, sem)
    cp.start()

    lane = lax.broadcasted_iota(jnp.int32, (1, 128), 1)
    for qi in range(_Q):
        qidx = q_sm[b * _Q + qi]
        qv_s[pl.ds(qi, 1), :] = emb_ref[qidx]
        row = idf_ref[qidx >> 7]
        idfm_s[pl.ds(qi, 1), :] = row * (lane == (qidx & 127)).astype(jnp.float32)

    qv = qv_s[...]
    qn = jnp.sqrt(jnp.sum(qv * qv, axis=1, keepdims=True))
    q_idf = jnp.sum(idfm_s[...], axis=1, keepdims=True)

    z = wg_ref[...] * q_idf
    z = z - jnp.max(z, axis=0, keepdims=True)
    ez = jnp.exp(z)
    gate = ez / jnp.sum(ez, axis=0, keepdims=True)

    cp.wait()

    ones_row = jnp.ones((1, 128), jnp.float32)

    def per_passage(dd, carry):
        doff = dd * _L

        for u in range(_L):
            tile_s[pl.ds(u, 1), :] = emb_ref[doc_sm[0, doff + u]]

        tile = tile_s[...]
        dn_dims = (((1,), (1,)), ((), ()))
        dots = lax.dot_general(qv, tile, dn_dims,
                               preferred_element_type=jnp.float32)
        sq = tile * tile
        n2 = jnp.sum(sq.reshape(4, 128, _E), axis=2).reshape(1, _L)
        denom = qn * jnp.sqrt(n2) + 1e-8
        sims = dots / denom

        bf = jnp.floor((sims - _LO) / (_HI - _LO) * _NB)
        bf = jnp.clip(bf, 0.0, float(_NB - 1))

        for nb in range(_NB):
            cnt = jnp.sum((bf == float(nb)).astype(jnp.float32),
                          axis=1, keepdims=True)
            x_s[pl.ds(dd * _Q, _Q), pl.ds(nb, 1)] = cnt
        return carry

    lax.fori_loop(0, _D, per_passage, 0)

    x = x_s[...]
    x = gin_ref[...] * (x * _BN_INV) + bin_ref[...]
    for w_r, b_r, g_r, be_r in ((w0_ref, b0_ref, g0_ref, be0_ref),
                                (w1_ref, b1_ref, g1_ref, be1_ref),
                                (w2_ref, b2_ref, g2_ref, be2_ref),
                                (w3_ref, b3_ref, g3_ref, be3_ref)):
        y = jnp.dot(x, w_r[...], preferred_element_type=jnp.float32) + b_r[...]
        x = jnp.tanh(g_r[...] * (y * _BN_INV) + be_r[...])

    qout = x
    gate_t = jnp.concatenate([gate] * _D, axis=0)
    prod = qout * gate_t
    si = lax.broadcasted_iota(jnp.int32, (128, _D), 0)
    ci = lax.broadcasted_iota(jnp.int32, (128, _D), 1)
    pm = ((si >> 3) == ci).astype(jnp.float32)
    out = lax.dot_general(prod, pm, (((0,), (0,)), ((), ())),
                          preferred_element_type=jnp.float32)
    out_ref[0] = out


def _whole(shape):
    zeros = (0,) * len(shape)
    return pl.BlockSpec(shape, lambda b, *_: zeros)


@jax.jit
def kernel(doc, query, item_embedding, idf_table, w_gate, g_in, b_in,
           W0, b0, g0, be0, W1, b1, g1, be1, W2, b2, g2, be2, W3, b3, g3, be3):
    emb3 = item_embedding.reshape(_NI, 1, _E)
    idf_flat = jnp.pad(idf_table[:, 0], (0, _IDF_ROWS * 128 - _NI))
    idf3 = idf_flat.reshape(_IDF_ROWS, 1, 128)
    doc3 = doc.reshape(_B, 1, _D * _L)
    qflat = query.reshape(-1)

    row1 = lambda a: a.reshape(1, -1)
    params = (emb3, idf3, doc3, w_gate.reshape(_Q, 1),
              row1(g_in), row1(b_in),
              W0, row1(b0), row1(g0), row1(be0),
              W1, row1(b1), row1(g1), row1(be1),
              W2, row1(b2), row1(g2), row1(be2),
              W3, row1(b3), row1(g3), row1(be3))

    in_specs = [
        _whole((_NI, 1, _E)),
        _whole((_IDF_ROWS, 1, 128)),
        pl.BlockSpec((1, 1, _D * _L), lambda b, *_: (b, 0, 0)),
    ] + [_whole(p.shape) for p in params[3:]]

    grid_spec = pltpu.PrefetchScalarGridSpec(
        num_scalar_prefetch=1,
        grid=(_B,),
        in_specs=in_specs,
        out_specs=pl.BlockSpec((1, 1, _D), lambda b, *_: (b, 0, 0)),
        scratch_shapes=[
            pltpu.SMEM((1, _D * _L), jnp.int32),
            pltpu.VMEM((_Q, _E), jnp.float32),
            pltpu.VMEM((_Q, _E), jnp.float32),
            pltpu.VMEM((_L, _E), jnp.float32),
            pltpu.VMEM((_D * _Q, _NB), jnp.float32),
            pltpu.SemaphoreType.DMA,
        ],
    )

    out = pl.pallas_call(
        _drmm_kernel,
        grid_spec=grid_spec,
        out_shape=jax.ShapeDtypeStruct((_B, 1, _D), jnp.float32),
        compiler_params=pltpu.CompilerParams(
            dimension_semantics=("parallel",),
            vmem_limit_bytes=56 * 1024 * 1024,
        ),
    )(qflat, *params)
    return out.reshape(_B, _D)


# speedup vs baseline: 1.6577x; 1.1184x over previous
import functools

import jax
import jax.numpy as jnp
import numpy as np
from jax import lax
from jax.experimental import pallas as pl
from jax.experimental.pallas import tpu as pltpu

_B, _D, _L, _Q, _E, _NB, _NI = 32, 16, 512, 8, 128, 30, 100000
_BN_EPS = 1e-3
_LO, _HI = 0.001, 1.0
_BN_INV = np.float32(1.0 / np.sqrt(1.0 + _BN_EPS))
_IDF_ROWS = (_NI + 127) // 128


def _drmm_kernel(q_sm, emb_ref, idf_ref, doc_ref, wg_ref, gin_ref, bin_ref,
                 w0_ref, b0_ref, g0_ref, be0_ref,
                 w1_ref, b1_ref, g1_ref, be1_ref,
                 w2_ref, b2_ref, g2_ref, be2_ref,
                 w3_ref, b3_ref, g3_ref, be3_ref,
                 out_ref,
                 doc_sm, qv_s, idfm_s, tile_a, tile_b, x_s, sem):
    b = pl.program_id(0)

    cp = pltpu.make_async_copy(doc_ref.at[0], doc_sm, sem)
    cp.start()

    lane = lax.broadcasted_iota(jnp.int32, (1, 128), 1)
    for qi in range(_Q):
        qidx = q_sm[b * _Q + qi]
        qv_s[pl.ds(qi, 1), :] = emb_ref[qidx]
        row = idf_ref[qidx >> 7]
        idfm_s[pl.ds(qi, 1), :] = row * (lane == (qidx & 127)).astype(jnp.float32)

    qv = qv_s[...]
    qn = jnp.sqrt(jnp.sum(qv * qv, axis=1, keepdims=True))
    q_idf = jnp.sum(idfm_s[...], axis=1, keepdims=True)

    z = wg_ref[...] * q_idf
    z = z - jnp.max(z, axis=0, keepdims=True)
    ez = jnp.exp(z)
    gate = ez / jnp.sum(ez, axis=0, keepdims=True)

    cp.wait()

    ones_row = jnp.ones((1, 128), jnp.float32)

    def gather_to(t_ref, dd):
        doff = dd * _L
        for u in range(_L):
            t_ref[pl.ds(u, 1), :] = emb_ref[doc_sm[0, doff + u]]

    def compute_d(t_ref, dd):
        tile = t_ref[...]
        dn_dims = (((1,), (1,)), ((), ()))
        dots = lax.dot_general(qv, tile, dn_dims,
                               preferred_element_type=jnp.float32)
        sq = tile * tile
        n2 = jnp.sum(sq.reshape(4, 128, _E), axis=2).reshape(1, _L)
        denom = qn * jnp.sqrt(n2) + 1e-8
        sims = dots / denom

        bf = jnp.floor((sims - _LO) / (_HI - _LO) * _NB)
        bf = jnp.clip(bf, 0.0, float(_NB - 1))

        for nb in range(_NB):
            cnt = jnp.sum((bf == float(nb)).astype(jnp.float32),
                          axis=1, keepdims=True)
            x_s[pl.ds(dd * _Q, _Q), pl.ds(nb, 1)] = cnt

    gather_to(tile_a, 0)

    def pair_body(k, carry):
        d0 = 2 * k
        gather_to(tile_b, d0 + 1)
        compute_d(tile_a, d0)

        @pl.when(k < _D // 2 - 1)
        def _():
            gather_to(tile_a, d0 + 2)

        compute_d(tile_b, d0 + 1)
        return carry

    lax.fori_loop(0, _D // 2, pair_body, 0)

    x = x_s[...]
    x = gin_ref[...] * (x * _BN_INV) + bin_ref[...]
    for w_r, b_r, g_r, be_r in ((w0_ref, b0_ref, g0_ref, be0_ref),
                                (w1_ref, b1_ref, g1_ref, be1_ref),
                                (w2_ref, b2_ref, g2_ref, be2_ref),
                                (w3_ref, b3_ref, g3_ref, be3_ref)):
        y = jnp.dot(x, w_r[...], preferred_element_type=jnp.float32) + b_r[...]
        x = jnp.tanh(g_r[...] * (y * _BN_INV) + be_r[...])

    qout = x
    gate_t = jnp.concatenate([gate] * _D, axis=0)
    prod = qout * gate_t
    si = lax.broadcasted_iota(jnp.int32, (128, _D), 0)
    ci = lax.broadcasted_iota(jnp.int32, (128, _D), 1)
    pm = ((si >> 3) == ci).astype(jnp.float32)
    out = lax.dot_general(prod, pm, (((0,), (0,)), ((), ())),
                          preferred_element_type=jnp.float32)
    out_ref[0] = out


def _whole(shape):
    zeros = (0,) * len(shape)
    return pl.BlockSpec(shape, lambda b, *_: zeros)


@jax.jit
def kernel(doc, query, item_embedding, idf_table, w_gate, g_in, b_in,
           W0, b0, g0, be0, W1, b1, g1, be1, W2, b2, g2, be2, W3, b3, g3, be3):
    emb3 = item_embedding.reshape(_NI, 1, _E)
    idf_flat = jnp.pad(idf_table[:, 0], (0, _IDF_ROWS * 128 - _NI))
    idf3 = idf_flat.reshape(_IDF_ROWS, 1, 128)
    doc3 = doc.reshape(_B, 1, _D * _L)
    qflat = query.reshape(-1)

    row1 = lambda a: a.reshape(1, -1)
    params = (emb3, idf3, doc3, w_gate.reshape(_Q, 1),
              row1(g_in), row1(b_in),
              W0, row1(b0), row1(g0), row1(be0),
              W1, row1(b1), row1(g1), row1(be1),
              W2, row1(b2), row1(g2), row1(be2),
              W3, row1(b3), row1(g3), row1(be3))

    in_specs = [
        _whole((_NI, 1, _E)),
        _whole((_IDF_ROWS, 1, 128)),
        pl.BlockSpec((1, 1, _D * _L), lambda b, *_: (b, 0, 0)),
    ] + [_whole(p.shape) for p in params[3:]]

    grid_spec = pltpu.PrefetchScalarGridSpec(
        num_scalar_prefetch=1,
        grid=(_B,),
        in_specs=in_specs,
        out_specs=pl.BlockSpec((1, 1, _D), lambda b, *_: (b, 0, 0)),
        scratch_shapes=[
            pltpu.SMEM((1, _D * _L), jnp.int32),
            pltpu.VMEM((_Q, _E), jnp.float32),
            pltpu.VMEM((_Q, _E), jnp.float32),
            pltpu.VMEM((_L, _E), jnp.float32),
            pltpu.VMEM((_L, _E), jnp.float32),
            pltpu.VMEM((_D * _Q, _NB), jnp.float32),
            pltpu.SemaphoreType.DMA,
        ],
    )

    out = pl.pallas_call(
        _drmm_kernel,
        grid_spec=grid_spec,
        out_shape=jax.ShapeDtypeStruct((_B, 1, _D), jnp.float32),
        compiler_params=pltpu.CompilerParams(
            dimension_semantics=("parallel",),
            vmem_limit_bytes=56 * 1024 * 1024,
        ),
    )(qflat, *params)
    return out.reshape(_B, _D)


# speedup vs baseline: 1.7261x; 1.0413x over previous
import functools

import jax
import jax.numpy as jnp
import numpy as np
from jax import lax
from jax.experimental import pallas as pl
from jax.experimental.pallas import tpu as pltpu

_B, _D, _L, _Q, _E, _NB, _NI = 32, 16, 512, 8, 128, 30, 100000
_BN_EPS = 1e-3
_LO, _HI = 0.001, 1.0
_BN_INV = np.float32(1.0 / np.sqrt(1.0 + _BN_EPS))
_IDF_ROWS = (_NI + 127) // 128


def _drmm_kernel(q_sm, emb_ref, idf_ref, doc_ref, wg_ref, gin_ref, bin_ref,
                 w0_ref, b0_ref, g0_ref, be0_ref,
                 w1_ref, b1_ref, g1_ref, be1_ref,
                 w2_ref, b2_ref, g2_ref, be2_ref,
                 w3_ref, b3_ref, g3_ref, be3_ref,
                 out_ref,
                 doc_sm, qv_s, idfm_s, tile_a, tile_b, x_s, sem):
    b = pl.program_id(0)

    cp = pltpu.make_async_copy(doc_ref.at[0], doc_sm, sem)
    cp.start()

    lane = lax.broadcasted_iota(jnp.int32, (1, 128), 1)
    for qi in range(_Q):
        qidx = q_sm[b * _Q + qi]
        qv_s[pl.ds(qi, 1), :] = emb_ref[qidx]
        row = idf_ref[qidx >> 7]
        idfm_s[pl.ds(qi, 1), :] = row * (lane == (qidx & 127)).astype(jnp.float32)

    qv = qv_s[...]
    qn = jnp.sqrt(jnp.sum(qv * qv, axis=1, keepdims=True))
    q_idf = jnp.sum(idfm_s[...], axis=1, keepdims=True)

    z = wg_ref[...] * q_idf
    z = z - jnp.max(z, axis=0, keepdims=True)
    ez = jnp.exp(z)
    gate = ez / jnp.sum(ez, axis=0, keepdims=True)

    cp.wait()

    ones_row = jnp.ones((1, 128), jnp.float32)

    def gather_to(t_ref, dd):
        doff = dd * _L
        for u in range(_L):
            t_ref[pl.ds(u, 1), :] = emb_ref[doc_sm[0, doff + u]]

    def compute_d(t_ref, dd):
        tile = t_ref[...]
        dn_dims = (((1,), (1,)), ((), ()))
        dots = lax.dot_general(qv, tile, dn_dims,
                               preferred_element_type=jnp.float32)
        sq = tile * tile
        n2 = jnp.sum(sq.reshape(4, 128, _E), axis=2).reshape(1, _L)
        denom = qn * jnp.sqrt(n2) + 1e-8
        sims = dots / denom

        bf = jnp.floor((sims - _LO) / (_HI - _LO) * _NB)
        bf = jnp.clip(bf, 0.0, float(_NB - 1))

        for nb in range(_NB):
            cnt = jnp.sum((bf == float(nb)).astype(jnp.float32),
                          axis=1, keepdims=True)
            x_s[pl.ds(dd * _Q, _Q), pl.ds(nb, 1)] = cnt

    gather_to(tile_a, 0)

    def pair_body(k, carry):
        d0 = 2 * k
        gather_to(tile_b, d0 + 1)
        compute_d(tile_a, d0)

        gather_to(tile_a, jnp.minimum(d0 + 2, _D - 1))

        compute_d(tile_b, d0 + 1)
        return carry

    lax.fori_loop(0, _D // 2, pair_body, 0)

    x = x_s[...]
    x = gin_ref[...] * (x * _BN_INV) + bin_ref[...]
    for w_r, b_r, g_r, be_r in ((w0_ref, b0_ref, g0_ref, be0_ref),
                                (w1_ref, b1_ref, g1_ref, be1_ref),
                                (w2_ref, b2_ref, g2_ref, be2_ref),
                                (w3_ref, b3_ref, g3_ref, be3_ref)):
        y = jnp.dot(x, w_r[...], preferred_element_type=jnp.float32) + b_r[...]
        x = jnp.tanh(g_r[...] * (y * _BN_INV) + be_r[...])

    qout = x
    gate_t = jnp.concatenate([gate] * _D, axis=0)
    prod = qout * gate_t
    si = lax.broadcasted_iota(jnp.int32, (128, _D), 0)
    ci = lax.broadcasted_iota(jnp.int32, (128, _D), 1)
    pm = ((si >> 3) == ci).astype(jnp.float32)
    out = lax.dot_general(prod, pm, (((0,), (0,)), ((), ())),
                          preferred_element_type=jnp.float32)
    out_ref[0] = out


def _whole(shape):
    zeros = (0,) * len(shape)
    return pl.BlockSpec(shape, lambda b, *_: zeros)


@jax.jit
def kernel(doc, query, item_embedding, idf_table, w_gate, g_in, b_in,
           W0, b0, g0, be0, W1, b1, g1, be1, W2, b2, g2, be2, W3, b3, g3, be3):
    emb3 = item_embedding.reshape(_NI, 1, _E)
    idf_flat = jnp.pad(idf_table[:, 0], (0, _IDF_ROWS * 128 - _NI))
    idf3 = idf_flat.reshape(_IDF_ROWS, 1, 128)
    doc3 = doc.reshape(_B, 1, _D * _L)
    qflat = query.reshape(-1)

    row1 = lambda a: a.reshape(1, -1)
    params = (emb3, idf3, doc3, w_gate.reshape(_Q, 1),
              row1(g_in), row1(b_in),
              W0, row1(b0), row1(g0), row1(be0),
              W1, row1(b1), row1(g1), row1(be1),
              W2, row1(b2), row1(g2), row1(be2),
              W3, row1(b3), row1(g3), row1(be3))

    in_specs = [
        _whole((_NI, 1, _E)),
        _whole((_IDF_ROWS, 1, 128)),
        pl.BlockSpec((1, 1, _D * _L), lambda b, *_: (b, 0, 0)),
    ] + [_whole(p.shape) for p in params[3:]]

    grid_spec = pltpu.PrefetchScalarGridSpec(
        num_scalar_prefetch=1,
        grid=(_B,),
        in_specs=in_specs,
        out_specs=pl.BlockSpec((1, 1, _D), lambda b, *_: (b, 0, 0)),
        scratch_shapes=[
            pltpu.SMEM((1, _D * _L), jnp.int32),
            pltpu.VMEM((_Q, _E), jnp.float32),
            pltpu.VMEM((_Q, _E), jnp.float32),
            pltpu.VMEM((_L, _E), jnp.float32),
            pltpu.VMEM((_L, _E), jnp.float32),
            pltpu.VMEM((_D * _Q, _NB), jnp.float32),
            pltpu.SemaphoreType.DMA,
        ],
    )

    out = pl.pallas_call(
        _drmm_kernel,
        grid_spec=grid_spec,
        out_shape=jax.ShapeDtypeStruct((_B, 1, _D), jnp.float32),
        compiler_params=pltpu.CompilerParams(
            dimension_semantics=("parallel",),
            vmem_limit_bytes=56 * 1024 * 1024,
        ),
    )(qflat, *params)
    return out.reshape(_B, _D)


# speedup vs baseline: 1.7286x; 1.0014x over previous
import jax
import jax.numpy as jnp
import numpy as np
from jax import lax
from jax.experimental import pallas as pl
from jax.experimental.pallas import tpu as pltpu

_B, _D, _L, _Q, _E, _NB, _NI = 32, 16, 512, 8, 128, 30, 100000
_BN_EPS = 1e-3
_LO, _HI = 0.001, 1.0
_BN_INV = np.float32(1.0 / np.sqrt(1.0 + _BN_EPS))
_IDF_ROWS = (_NI + 127) // 128


def _drmm_kernel(q_sm, emb_ref, idf_ref, doc_ref, wg_ref, gin_ref, bin_ref,
                 w0_ref, b0_ref, g0_ref, be0_ref,
                 w1_ref, b1_ref, g1_ref, be1_ref,
                 w2_ref, b2_ref, g2_ref, be2_ref,
                 w3_ref, b3_ref, g3_ref, be3_ref,
                 out_ref,
                 doc_sm, qv_s, idfm_s, tile_a, tile_b, x_s, sem):
    b = pl.program_id(0)

    cp = pltpu.make_async_copy(doc_ref.at[0], doc_sm, sem)
    cp.start()

    lane = lax.broadcasted_iota(jnp.int32, (1, 128), 1)
    for qi in range(_Q):
        qidx = q_sm[b * _Q + qi]
        qv_s[pl.ds(qi, 1), :] = emb_ref[qidx]
        row = idf_ref[qidx >> 7]
        idfm_s[pl.ds(qi, 1), :] = row * (lane == (qidx & 127)).astype(jnp.float32)

    qv = qv_s[...]
    qn = jnp.sqrt(jnp.sum(qv * qv, axis=1, keepdims=True))
    q_idf = jnp.sum(idfm_s[...], axis=1, keepdims=True)

    z = wg_ref[...] * q_idf
    z = z - jnp.max(z, axis=0, keepdims=True)
    ez = jnp.exp(z)
    gate = ez / jnp.sum(ez, axis=0, keepdims=True)

    cp.wait()

    ones_row = jnp.ones((1, 128), jnp.float32)

    def gather_to(t_ref, dd):
        doff = dd * _L
        for u in range(_L):
            t_ref[pl.ds(u, 1), :] = emb_ref[doc_sm[0, doff + u]]

    def compute_d(t_ref, dd):
        tile = t_ref[...]
        dn_dims = (((1,), (1,)), ((), ()))
        dots = lax.dot_general(qv, tile, dn_dims,
                               preferred_element_type=jnp.float32)
        sq = tile * tile
        n2 = jnp.sum(sq.reshape(4, 128, _E), axis=2).reshape(1, _L)
        denom = qn * jnp.sqrt(n2) + 1e-8
        sims = dots / denom

        bf = jnp.floor((sims - _LO) / (_HI - _LO) * _NB)
        bf = jnp.clip(bf, 0.0, float(_NB - 1))

        for nb in range(_NB):
            cnt = jnp.sum((bf == float(nb)).astype(jnp.float32),
                          axis=1, keepdims=True)
            x_s[pl.ds(dd * _Q, _Q), pl.ds(nb, 1)] = cnt

    gather_to(tile_a, 0)

    def pair_body(k, carry):
        d0 = 2 * k
        gather_to(tile_b, d0 + 1)
        compute_d(tile_a, d0)

        gather_to(tile_a, jnp.minimum(d0 + 2, _D - 1))

        compute_d(tile_b, d0 + 1)
        return carry

    lax.fori_loop(0, _D // 2, pair_body, 0)

    x = x_s[...]
    x = gin_ref[...] * (x * _BN_INV) + bin_ref[...]
    for w_r, b_r, g_r, be_r in ((w0_ref, b0_ref, g0_ref, be0_ref),
                                (w1_ref, b1_ref, g1_ref, be1_ref),
                                (w2_ref, b2_ref, g2_ref, be2_ref),
                                (w3_ref, b3_ref, g3_ref, be3_ref)):
        y = jnp.dot(x, w_r[...], preferred_element_type=jnp.float32) + b_r[...]
        x = jnp.tanh(g_r[...] * (y * _BN_INV) + be_r[...])

    qout = x
    gate_t = jnp.concatenate([gate] * _D, axis=0)
    prod = qout * gate_t
    si = lax.broadcasted_iota(jnp.int32, (128, _D), 0)
    ci = lax.broadcasted_iota(jnp.int32, (128, _D), 1)
    pm = ((si >> 3) == ci).astype(jnp.float32)
    out = lax.dot_general(prod, pm, (((0,), (0,)), ((), ())),
                          preferred_element_type=jnp.float32)
    out_ref[0] = out


def _whole(shape):
    zeros = (0,) * len(shape)
    return pl.BlockSpec(shape, lambda b, *_: zeros)


@jax.jit
def kernel(doc, query, item_embedding, idf_table, w_gate, g_in, b_in,
           W0, b0, g0, be0, W1, b1, g1, be1, W2, b2, g2, be2, W3, b3, g3, be3):
    emb3 = item_embedding.reshape(_NI, 1, _E)
    idf_flat = jnp.pad(idf_table[:, 0], (0, _IDF_ROWS * 128 - _NI))
    idf3 = idf_flat.reshape(_IDF_ROWS, 1, 128)
    doc3 = doc.reshape(_B, 1, _D * _L)
    qflat = query.reshape(-1)

    row1 = lambda a: a.reshape(1, -1)
    params = (emb3, idf3, doc3, w_gate.reshape(_Q, 1),
              row1(g_in), row1(b_in),
              W0, row1(b0), row1(g0), row1(be0),
              W1, row1(b1), row1(g1), row1(be1),
              W2, row1(b2), row1(g2), row1(be2),
              W3, row1(b3), row1(g3), row1(be3))

    in_specs = [
        _whole((_NI, 1, _E)),
        _whole((_IDF_ROWS, 1, 128)),
        pl.BlockSpec((1, 1, _D * _L), lambda b, *_: (b, 0, 0)),
    ] + [_whole(p.shape) for p in params[3:]]

    grid_spec = pltpu.PrefetchScalarGridSpec(
        num_scalar_prefetch=1,
        grid=(_B,),
        in_specs=in_specs,
        out_specs=pl.BlockSpec((1, 1, _D), lambda b, *_: (b, 0, 0)),
        scratch_shapes=[
            pltpu.SMEM((1, _D * _L), jnp.int32),
            pltpu.VMEM((_Q, _E), jnp.float32),
            pltpu.VMEM((_Q, _E), jnp.float32),
            pltpu.VMEM((_L, _E), jnp.float32),
            pltpu.VMEM((_L, _E), jnp.float32),
            pltpu.VMEM((_D * _Q, _NB), jnp.float32),
            pltpu.SemaphoreType.DMA,
        ],
    )

    out = pl.pallas_call(
        _drmm_kernel,
        grid_spec=grid_spec,
        out_shape=jax.ShapeDtypeStruct((_B, 1, _D), jnp.float32),
        compiler_params=pltpu.CompilerParams(
            dimension_semantics=("parallel",),
            vmem_limit_bytes=56 * 1024 * 1024,
        ),
    )(qflat, *params)
    return out.reshape(_B, _D)
